# baseline (speedup 1.0000x reference)
import functools

import jax
import jax.numpy as jnp
import numpy as np
from jax import lax
from jax.experimental import pallas as pl
from jax.experimental.pallas import tpu as pltpu
from jax.experimental.pallas import tpu_sc as plsc

E = 8
K = 2
D = 768
H = 512
T = 8192
EPS = float(np.finfo(float).eps)

BT = 1024
NBLK = T // BT
BM = 512
P = 2 * T + E * BM
NB = P // BM

NW = 32
CH = 64
TPW = T // NW


def _cumsum0(x):
    n = x.shape[0]
    shift = 1
    while shift < n:
        z = jnp.zeros((shift,) + x.shape[1:], x.dtype)
        x = x + jnp.concatenate([z, x[:-shift]], axis=0)
        shift *= 2
    return x


def _cumsum1(x):
    n = x.shape[1]
    shift = 1
    while shift < n:
        z = jnp.zeros(x.shape[:1] + (shift,) + x.shape[2:], x.dtype)
        x = x + jnp.concatenate([z, x[:, :-shift]], axis=1)
        shift *= 2
    return x


def _k1_body(f_ref, gW1_ref, gb1_ref, gW2_ref, gb2_ref,
             g1_ref, g2_ref, e1_ref, e2_ref, r1_ref, r2_ref,
             cnt_ref, imp_ref, loss_ref):
    i = pl.program_id(0)
    nsteps = pl.num_programs(0)

    f = f_ref[...]
    h = jnp.maximum(
        jnp.dot(f, gW1_ref[...], preferred_element_type=jnp.float32)
        + gb1_ref[...], 0.0)
    logits = (jnp.dot(h, gW2_ref[...], preferred_element_type=jnp.float32)
              + gb2_ref[...])

    lane = lax.broadcasted_iota(jnp.int32, logits.shape, 1)
    m1 = jnp.max(logits, axis=1, keepdims=True)
    a1 = jnp.argmax(logits, axis=1).astype(jnp.int32)[:, None]
    masked = jnp.where(lane == a1, -jnp.inf, logits)
    m2 = jnp.max(masked, axis=1, keepdims=True)
    a2 = jnp.argmax(masked, axis=1).astype(jnp.int32)[:, None]
    e2v = jnp.exp(m2 - m1)
    denom = 1.0 + e2v
    g1 = 1.0 / denom
    g2 = e2v / denom

    g1_ref[...] = g1
    g2_ref[...] = g2
    e1_ref[...] = a1
    e2_ref[...] = a2

    oh = (jnp.where(lane == a1, 1, 0) + jnp.where(lane == a2, 1, 0)
          ).astype(jnp.int32)
    cume = _cumsum0(oh) - oh
    r1_ref[...] = jnp.sum(jnp.where(lane == a1, cume, 0), axis=1,
                          keepdims=True)
    r2_ref[...] = jnp.sum(jnp.where(lane == a2, cume, 0), axis=1,
                          keepdims=True)
    cnt_ref[...] = jnp.sum(oh, axis=0, keepdims=True)[None]

    gates = (jnp.where(lane == a1, g1, 0.0)
             + jnp.where(lane == a2, g2, 0.0))
    imp_blk = jnp.sum(gates, axis=0, keepdims=True)
    load_blk = jnp.sum((gates > 0.0).astype(jnp.float32), axis=0,
                       keepdims=True)
    blk = jnp.concatenate([imp_blk, load_blk], axis=0)

    @pl.when(i == 0)
    def _init():
        imp_ref[...] = blk

    @pl.when(i > 0)
    def _acc():
        imp_ref[...] += blk

    @pl.when(i == nsteps - 1)
    def _loss():
        acc = imp_ref[...]
        mean = jnp.mean(acc, axis=1, keepdims=True)
        var = jnp.sum((acc - mean) ** 2, axis=1, keepdims=True) / (E - 1)
        cv2 = var / (mean * mean + 1e-10)
        loss_ref[...] = (cv2[0:1, 0:1] + cv2[1:2, 0:1]) * 1e-2


def _k2_body(cnt_ref, base_ref, be_ref):
    c = cnt_ref[...].reshape(NBLK, E)
    cnt = jnp.sum(c, axis=0, keepdims=True)
    cnt_pad = ((cnt + BM - 1) // BM) * BM
    seg_start = _cumsum1(cnt_pad) - cnt_pad
    blk_excl = _cumsum0(c) - c
    base_ref[...] = (seg_start + blk_excl).reshape(NBLK, 1, E)
    rows = lax.broadcasted_iota(jnp.int32, (NB, E), 0) * BM
    ge = (rows >= seg_start).astype(jnp.int32)
    be_ref[...] = jnp.sum(ge, axis=1) - 1


def _k3_body(e1_ref, e2_ref, r1_ref, r2_ref, base_ref, p1_ref, p2_ref):
    base_row = base_ref[0]
    shp = (e1_ref.shape[0], E)
    lane = lax.broadcasted_iota(jnp.int32, shp, 1)
    b = jnp.broadcast_to(base_row, shp)
    base1 = jnp.sum(jnp.where(lane == e1_ref[...], b, 0), axis=1,
                    keepdims=True)
    base2 = jnp.sum(jnp.where(lane == e2_ref[...], b, 0), axis=1,
                    keepdims=True)
    p1_ref[...] = base1 + r1_ref[...]
    p2_ref[...] = base2 + r2_ref[...]


def _k5_body(be_ref, xs_ref, aW_ref, ab_ref, ys_ref):
    z = (jnp.dot(xs_ref[...], aW_ref[0], preferred_element_type=jnp.float32)
         + ab_ref[0])
    ys_ref[...] = jnp.exp(z)


def _k4_sc(x_hbm, p1_hbm, p2_hbm, xs_hbm, idx1_v, idx2_v, xbuf, sem):
    wid = lax.axis_index("s") * 2 + lax.axis_index("c")
    for c in range(TPW // CH):
        base = wid * TPW + c * CH
        pltpu.sync_copy(p1_hbm.at[pl.ds(base, CH)], idx1_v)
        pltpu.sync_copy(p2_hbm.at[pl.ds(base, CH)], idx2_v)
        pltpu.sync_copy(x_hbm.at[pl.ds(base, CH)], xbuf)
        pltpu.async_copy(xbuf, xs_hbm.at[idx1_v], sem).wait()
        pltpu.async_copy(xbuf, xs_hbm.at[idx2_v], sem).wait()


def _k6_sc(ys_hbm, p1_hbm, p2_hbm, r1_hbm, r2_hbm,
           idx1_v, idx2_v, buf1, buf2, sem):
    wid = lax.axis_index("s") * 2 + lax.axis_index("c")
    for c in range(TPW // CH):
        base = wid * TPW + c * CH
        pltpu.sync_copy(p1_hbm.at[pl.ds(base, CH)], idx1_v)
        pltpu.sync_copy(p2_hbm.at[pl.ds(base, CH)], idx2_v)
        pltpu.async_copy(ys_hbm.at[idx1_v], buf1, sem).wait()
        pltpu.async_copy(ys_hbm.at[idx2_v], buf2, sem).wait()
        pltpu.sync_copy(buf1, r1_hbm.at[pl.ds(base, CH)])
        pltpu.sync_copy(buf2, r2_hbm.at[pl.ds(base, CH)])


def _k7_body(r1_ref, r2_ref, g1_ref, g2_ref, y_ref):
    s = g1_ref[...] * r1_ref[...] + g2_ref[...] * r2_ref[...]
    y_ref[...] = jnp.log(jnp.where(s == 0.0, EPS, s))


def _gating(feature, gW1, gb1, gW2, gb2):
    return pl.pallas_call(
        _k1_body,
        grid=(NBLK,),
        in_specs=[
            pl.BlockSpec((BT, D), lambda i: (i, 0)),
            pl.BlockSpec((D, H), lambda i: (0, 0)),
            pl.BlockSpec((H,), lambda i: (0,)),
            pl.BlockSpec((H, E), lambda i: (0, 0)),
            pl.BlockSpec((E,), lambda i: (0,)),
        ],
        out_specs=[
            pl.BlockSpec((BT, 1), lambda i: (i, 0)),
            pl.BlockSpec((BT, 1), lambda i: (i, 0)),
            pl.BlockSpec((BT, 1), lambda i: (i, 0)),
            pl.BlockSpec((BT, 1), lambda i: (i, 0)),
            pl.BlockSpec((BT, 1), lambda i: (i, 0)),
            pl.BlockSpec((BT, 1), lambda i: (i, 0)),
            pl.BlockSpec((1, 1, E), lambda i: (i, 0, 0)),
            pl.BlockSpec((2, E), lambda i: (0, 0)),
            pl.BlockSpec((1, 1), lambda i: (0, 0)),
        ],
        out_shape=[
            jax.ShapeDtypeStruct((T, 1), jnp.float32),
            jax.ShapeDtypeStruct((T, 1), jnp.float32),
            jax.ShapeDtypeStruct((T, 1), jnp.int32),
            jax.ShapeDtypeStruct((T, 1), jnp.int32),
            jax.ShapeDtypeStruct((T, 1), jnp.int32),
            jax.ShapeDtypeStruct((T, 1), jnp.int32),
            jax.ShapeDtypeStruct((NBLK, 1, E), jnp.int32),
            jax.ShapeDtypeStruct((2, E), jnp.float32),
            jax.ShapeDtypeStruct((1, 1), jnp.float32),
        ],
        compiler_params=pltpu.CompilerParams(
            dimension_semantics=("arbitrary",)),
    )(feature, gW1, gb1, gW2, gb2)


def _offsets(cnt_blk):
    return pl.pallas_call(
        _k2_body,
        grid=(1,),
        in_specs=[pl.BlockSpec((NBLK, 1, E), lambda i: (0, 0, 0))],
        out_specs=[
            pl.BlockSpec((NBLK, 1, E), lambda i: (0, 0, 0)),
            pl.BlockSpec((NB,), lambda i: (0,)),
        ],
        out_shape=[
            jax.ShapeDtypeStruct((NBLK, 1, E), jnp.int32),
            jax.ShapeDtypeStruct((NB,), jnp.int32),
        ],
    )(cnt_blk)


def _positions(e1, e2, rank1, rank2, base_blk):
    return pl.pallas_call(
        _k3_body,
        grid=(NBLK,),
        in_specs=[
            pl.BlockSpec((BT, 1), lambda i: (i, 0)),
            pl.BlockSpec((BT, 1), lambda i: (i, 0)),
            pl.BlockSpec((BT, 1), lambda i: (i, 0)),
            pl.BlockSpec((BT, 1), lambda i: (i, 0)),
            pl.BlockSpec((1, 1, E), lambda i: (i, 0, 0)),
        ],
        out_specs=[
            pl.BlockSpec((BT, 1), lambda i: (i, 0)),
            pl.BlockSpec((BT, 1), lambda i: (i, 0)),
        ],
        out_shape=[
            jax.ShapeDtypeStruct((T, 1), jnp.int32),
            jax.ShapeDtypeStruct((T, 1), jnp.int32),
        ],
    )(e1, e2, rank1, rank2, base_blk)


def _dispatch(x, pos1, pos2):
    mesh = plsc.VectorSubcoreMesh(core_axis_name="c", subcore_axis_name="s")
    f = functools.partial(
        pl.kernel,
        mesh=mesh,
        out_type=jax.ShapeDtypeStruct((P, D), jnp.float32),
        scratch_types=[
            pltpu.VMEM((CH,), jnp.int32),
            pltpu.VMEM((CH,), jnp.int32),
            pltpu.VMEM((CH, D), jnp.float32),
            pltpu.SemaphoreType.DMA,
        ],
    )(_k4_sc)
    return f(x, pos1, pos2)


def _expert_mm(xs, aW, ab3, be):
    return pl.pallas_call(
        _k5_body,
        grid_spec=pltpu.PrefetchScalarGridSpec(
            num_scalar_prefetch=1,
            grid=(NB,),
            in_specs=[
                pl.BlockSpec((BM, D), lambda i, be: (i, 0)),
                pl.BlockSpec((1, D, D), lambda i, be: (be[i], 0, 0)),
                pl.BlockSpec((1, 1, D), lambda i, be: (be[i], 0, 0)),
            ],
            out_specs=pl.BlockSpec((BM, D), lambda i, be: (i, 0)),
        ),
        out_shape=jax.ShapeDtypeStruct((P, D), jnp.float32),
        compiler_params=pltpu.CompilerParams(
            dimension_semantics=("arbitrary",)),
    )(be, xs, aW, ab3)


def _combine_gather(ys, pos1, pos2):
    mesh = plsc.VectorSubcoreMesh(core_axis_name="c", subcore_axis_name="s")
    f = functools.partial(
        pl.kernel,
        mesh=mesh,
        out_type=[
            jax.ShapeDtypeStruct((T, D), jnp.float32),
            jax.ShapeDtypeStruct((T, D), jnp.float32),
        ],
        scratch_types=[
            pltpu.VMEM((CH,), jnp.int32),
            pltpu.VMEM((CH,), jnp.int32),
            pltpu.VMEM((CH, D), jnp.float32),
            pltpu.VMEM((CH, D), jnp.float32),
            pltpu.SemaphoreType.DMA,
        ],
    )(_k6_sc)
    return f(ys, pos1, pos2)


def _combine_log(r1, r2, g1, g2):
    return pl.pallas_call(
        _k7_body,
        grid=(NBLK,),
        in_specs=[
            pl.BlockSpec((BT, D), lambda i: (i, 0)),
            pl.BlockSpec((BT, D), lambda i: (i, 0)),
            pl.BlockSpec((BT, 1), lambda i: (i, 0)),
            pl.BlockSpec((BT, 1), lambda i: (i, 0)),
        ],
        out_specs=pl.BlockSpec((BT, D), lambda i: (i, 0)),
        out_shape=jax.ShapeDtypeStruct((T, D), jnp.float32),
        compiler_params=pltpu.CompilerParams(
            dimension_semantics=("arbitrary",)),
    )(r1, r2, g1, g2)


@jax.jit
def kernel(feature, x, gW1, gb1, gW2, gb2, aW, ab):
    g1, g2, e1, e2, rank1, rank2, cnt_blk, _imp, loss2d = _gating(
        feature, gW1, gb1, gW2, gb2)
    base_blk, be = _offsets(cnt_blk)
    pos1, pos2 = _positions(e1, e2, rank1, rank2, base_blk)
    p1f = pos1.reshape(T)
    p2f = pos2.reshape(T)
    xs = _dispatch(x, p1f, p2f)
    ys = _expert_mm(xs, aW, ab.reshape(E, 1, D), be)
    r1, r2 = _combine_gather(ys, p1f, p2f)
    y = _combine_log(r1, r2, g1, g2)
    return y, loss2d[0, 0]


# speedup vs baseline: 1.0360x; 1.0360x over previous
import functools

import jax
import jax.numpy as jnp
import numpy as np
from jax import lax
from jax.experimental import pallas as pl
from jax.experimental.pallas import tpu as pltpu
from jax.experimental.pallas import tpu_sc as plsc

E = 8
K = 2
D = 768
H = 512
T = 8192
EPS = float(np.finfo(float).eps)

BT = 1024
NBLK = T // BT
BM = 512
P = 2 * T + E * BM
NB = P // BM

NW = 32
TPW = T // NW
CHD = 64
NCHD = TPW // CHD
CHC = 32
NCHC = TPW // CHC


def _cumsum0(x):
    n = x.shape[0]
    shift = 1
    while shift < n:
        z = jnp.zeros((shift,) + x.shape[1:], x.dtype)
        x = x + jnp.concatenate([z, x[:-shift]], axis=0)
        shift *= 2
    return x


def _cumsum1(x):
    n = x.shape[1]
    shift = 1
    while shift < n:
        z = jnp.zeros(x.shape[:1] + (shift,) + x.shape[2:], x.dtype)
        x = x + jnp.concatenate([z, x[:, :-shift]], axis=1)
        shift *= 2
    return x


def _k1_body(f_ref, gW1_ref, gb1_ref, gW2_ref, gb2_ref,
             g1_ref, g2_ref, e1_ref, e2_ref, r1_ref, r2_ref,
             blkx_ref, segs_ref, be_ref, imp_ref, loss_ref,
             acc_ref):
    i = pl.program_id(0)
    nsteps = pl.num_programs(0)

    f = f_ref[...]
    h = jnp.maximum(
        jnp.dot(f, gW1_ref[...], preferred_element_type=jnp.float32)
        + gb1_ref[...], 0.0)
    logits = (jnp.dot(h, gW2_ref[...], preferred_element_type=jnp.float32)
              + gb2_ref[...])

    lane = lax.broadcasted_iota(jnp.int32, logits.shape, 1)
    m1 = jnp.max(logits, axis=1, keepdims=True)
    a1 = jnp.argmax(logits, axis=1).astype(jnp.int32)[:, None]
    masked = jnp.where(lane == a1, -jnp.inf, logits)
    m2 = jnp.max(masked, axis=1, keepdims=True)
    a2 = jnp.argmax(masked, axis=1).astype(jnp.int32)[:, None]
    e2v = jnp.exp(m2 - m1)
    denom = 1.0 + e2v
    g1_ref[...] = 1.0 / denom
    g2_ref[...] = e2v / denom
    e1_ref[...] = a1
    e2_ref[...] = a2

    oh = (jnp.where(lane == a1, 1, 0) + jnp.where(lane == a2, 1, 0)
          ).astype(jnp.int32)
    cume = _cumsum0(oh) - oh
    r1_ref[...] = jnp.sum(jnp.where(lane == a1, cume, 0), axis=1,
                          keepdims=True)
    r2_ref[...] = jnp.sum(jnp.where(lane == a2, cume, 0), axis=1,
                          keepdims=True)
    cnt_now = jnp.sum(oh, axis=0, keepdims=True)

    @pl.when(i == 0)
    def _init_acc():
        acc_ref[...] = jnp.zeros_like(acc_ref)

    blkx_ref[...] = acc_ref[...][None]
    acc_ref[...] += cnt_now

    gates = (jnp.where(lane == a1, g1_ref[...], 0.0)
             + jnp.where(lane == a2, g2_ref[...], 0.0))
    imp_blk = jnp.sum(gates, axis=0, keepdims=True)
    load_blk = jnp.sum((gates > 0.0).astype(jnp.float32), axis=0,
                       keepdims=True)
    blk = jnp.concatenate([imp_blk, load_blk], axis=0)

    @pl.when(i == 0)
    def _init():
        imp_ref[...] = blk

    @pl.when(i > 0)
    def _acc():
        imp_ref[...] += blk

    @pl.when(i == nsteps - 1)
    def _fin():
        cnt = acc_ref[...]
        cnt_pad = ((cnt + BM - 1) // BM) * BM
        seg_start = _cumsum1(cnt_pad) - cnt_pad
        segs_ref[...] = seg_start
        rows = lax.broadcasted_iota(jnp.int32, (NB, E), 0) * BM
        ge = (rows >= seg_start).astype(jnp.int32)
        be_ref[...] = jnp.sum(ge, axis=1) - 1

        a = imp_ref[...]
        mean = jnp.mean(a, axis=1, keepdims=True)
        var = jnp.sum((a - mean) ** 2, axis=1, keepdims=True) / (E - 1)
        cv2 = var / (mean * mean + 1e-10)
        loss_ref[...] = (cv2[0:1, 0:1] + cv2[1:2, 0:1]) * 1e-2


def _k3_body(e1_ref, e2_ref, r1_ref, r2_ref, blkx_ref, segs_ref,
             p1_ref, p2_ref):
    base_row = blkx_ref[0] + segs_ref[...]
    shp = (e1_ref.shape[0], E)
    lane = lax.broadcasted_iota(jnp.int32, shp, 1)
    b = jnp.broadcast_to(base_row, shp)
    base1 = jnp.sum(jnp.where(lane == e1_ref[...], b, 0), axis=1,
                    keepdims=True)
    base2 = jnp.sum(jnp.where(lane == e2_ref[...], b, 0), axis=1,
                    keepdims=True)
    p1_ref[...] = base1 + r1_ref[...]
    p2_ref[...] = base2 + r2_ref[...]


def _k5_body(be_ref, xs_ref, aW_ref, ab_ref, ys_ref):
    w16 = aW_ref[0].astype(jnp.bfloat16)
    x16 = xs_ref[...].astype(jnp.bfloat16)
    z = (jnp.dot(x16, w16, preferred_element_type=jnp.float32)
         + ab_ref[0])
    ys_ref[...] = jnp.exp(z)


def _k4_sc(x_hbm, p1_hbm, p2_hbm, xs_hbm,
           idx1_v, idx2_v, xb0, xb1, lsem0, lsem1, ssem):
    wid = lax.axis_index("s") * 2 + lax.axis_index("c")
    pltpu.sync_copy(p1_hbm.at[wid], idx1_v)
    pltpu.sync_copy(p2_hbm.at[wid], idx2_v)
    bufs = (xb0, xb1)
    lsems = (lsem0, lsem1)
    loads = [None, None]
    scats = [[], []]
    loads[0] = pltpu.async_copy(
        x_hbm.at[pl.ds(wid * TPW, CHD)], xb0, lsem0)
    for c in range(NCHD):
        b = c % 2
        nb = (c + 1) % 2
        if c + 1 < NCHD:
            for h in scats[nb]:
                h.wait()
            scats[nb] = []
            loads[nb] = pltpu.async_copy(
                x_hbm.at[pl.ds(wid * TPW + (c + 1) * CHD, CHD)],
                bufs[nb], lsems[nb])
        loads[b].wait()
        scats[b] = [
            pltpu.async_copy(bufs[b], xs_hbm.at[idx1_v.at[c]], ssem),
            pltpu.async_copy(bufs[b], xs_hbm.at[idx2_v.at[c]], ssem),
        ]
    for hs in scats:
        for h in hs:
            h.wait()


def _k6_sc(ys_hbm, p1_hbm, p2_hbm, r1_hbm, r2_hbm,
           idx1_v, idx2_v, b1a, b2a, b1b, b2b, gsem_a, gsem_b, wsem):
    wid = lax.axis_index("s") * 2 + lax.axis_index("c")
    pltpu.sync_copy(p1_hbm.at[wid], idx1_v)
    pltpu.sync_copy(p2_hbm.at[wid], idx2_v)
    bufs = ((b1a, b2a), (b1b, b2b))
    gsems = (gsem_a, gsem_b)
    gath = [None, None]
    wbs = [[], []]
    gath[0] = [
        pltpu.async_copy(ys_hbm.at[idx1_v.at[0]], b1a, gsem_a),
        pltpu.async_copy(ys_hbm.at[idx2_v.at[0]], b2a, gsem_a),
    ]
    for c in range(NCHC):
        b = c % 2
        nb = (c + 1) % 2
        if c + 1 < NCHC:
            for h in wbs[nb]:
                h.wait()
            wbs[nb] = []
            gath[nb] = [
                pltpu.async_copy(ys_hbm.at[idx1_v.at[c + 1]],
                                 bufs[nb][0], gsems[nb]),
                pltpu.async_copy(ys_hbm.at[idx2_v.at[c + 1]],
                                 bufs[nb][1], gsems[nb]),
            ]
        for h in gath[b]:
            h.wait()
        base = wid * TPW + c * CHC
        wbs[b] = [
            pltpu.async_copy(bufs[b][0], r1_hbm.at[pl.ds(base, CHC)], wsem),
            pltpu.async_copy(bufs[b][1], r2_hbm.at[pl.ds(base, CHC)], wsem),
        ]
    for hs in wbs:
        for h in hs:
            h.wait()


def _k7_body(r1_ref, r2_ref, g1_ref, g2_ref, y_ref):
    s = g1_ref[...] * r1_ref[...] + g2_ref[...] * r2_ref[...]
    y_ref[...] = jnp.log(jnp.where(s == 0.0, EPS, s))


def _gating(feature, gW1, gb1, gW2, gb2):
    return pl.pallas_call(
        _k1_body,
        grid=(NBLK,),
        in_specs=[
            pl.BlockSpec((BT, D), lambda i: (i, 0)),
            pl.BlockSpec((D, H), lambda i: (0, 0)),
            pl.BlockSpec((H,), lambda i: (0,)),
            pl.BlockSpec((H, E), lambda i: (0, 0)),
            pl.BlockSpec((E,), lambda i: (0,)),
        ],
        out_specs=[
            pl.BlockSpec((BT, 1), lambda i: (i, 0)),
            pl.BlockSpec((BT, 1), lambda i: (i, 0)),
            pl.BlockSpec((BT, 1), lambda i: (i, 0)),
            pl.BlockSpec((BT, 1), lambda i: (i, 0)),
            pl.BlockSpec((BT, 1), lambda i: (i, 0)),
            pl.BlockSpec((BT, 1), lambda i: (i, 0)),
            pl.BlockSpec((1, 1, E), lambda i: (i, 0, 0)),
            pl.BlockSpec((1, E), lambda i: (0, 0)),
            pl.BlockSpec((NB,), lambda i: (0,)),
            pl.BlockSpec((2, E), lambda i: (0, 0)),
            pl.BlockSpec((1, 1), lambda i: (0, 0)),
        ],
        out_shape=[
            jax.ShapeDtypeStruct((T, 1), jnp.float32),
            jax.ShapeDtypeStruct((T, 1), jnp.float32),
            jax.ShapeDtypeStruct((T, 1), jnp.int32),
            jax.ShapeDtypeStruct((T, 1), jnp.int32),
            jax.ShapeDtypeStruct((T, 1), jnp.int32),
            jax.ShapeDtypeStruct((T, 1), jnp.int32),
            jax.ShapeDtypeStruct((NBLK, 1, E), jnp.int32),
            jax.ShapeDtypeStruct((1, E), jnp.int32),
            jax.ShapeDtypeStruct((NB,), jnp.int32),
            jax.ShapeDtypeStruct((2, E), jnp.float32),
            jax.ShapeDtypeStruct((1, 1), jnp.float32),
        ],
        scratch_shapes=[pltpu.VMEM((1, E), jnp.int32)],
        compiler_params=pltpu.CompilerParams(
            dimension_semantics=("arbitrary",)),
    )(feature, gW1, gb1, gW2, gb2)


def _positions(e1, e2, rank1, rank2, blk_excl, seg_start):
    return pl.pallas_call(
        _k3_body,
        grid=(NBLK,),
        in_specs=[
            pl.BlockSpec((BT, 1), lambda i: (i, 0)),
            pl.BlockSpec((BT, 1), lambda i: (i, 0)),
            pl.BlockSpec((BT, 1), lambda i: (i, 0)),
            pl.BlockSpec((BT, 1), lambda i: (i, 0)),
            pl.BlockSpec((1, 1, E), lambda i: (i, 0, 0)),
            pl.BlockSpec((1, E), lambda i: (0, 0)),
        ],
        out_specs=[
            pl.BlockSpec((BT, 1), lambda i: (i, 0)),
            pl.BlockSpec((BT, 1), lambda i: (i, 0)),
        ],
        out_shape=[
            jax.ShapeDtypeStruct((T, 1), jnp.int32),
            jax.ShapeDtypeStruct((T, 1), jnp.int32),
        ],
    )(e1, e2, rank1, rank2, blk_excl, seg_start)


def _dispatch(x, pos1, pos2):
    mesh = plsc.VectorSubcoreMesh(core_axis_name="c", subcore_axis_name="s")
    f = functools.partial(
        pl.kernel,
        mesh=mesh,
        out_type=jax.ShapeDtypeStruct((P, D), jnp.float32),
        scratch_types=[
            pltpu.VMEM((NCHD, CHD), jnp.int32),
            pltpu.VMEM((NCHD, CHD), jnp.int32),
            pltpu.VMEM((CHD, D), jnp.float32),
            pltpu.VMEM((CHD, D), jnp.float32),
            pltpu.SemaphoreType.DMA,
            pltpu.SemaphoreType.DMA,
            pltpu.SemaphoreType.DMA,
        ],
    )(_k4_sc)
    return f(x, pos1, pos2)


def _expert_mm(xs, aW, ab3, be):
    return pl.pallas_call(
        _k5_body,
        grid_spec=pltpu.PrefetchScalarGridSpec(
            num_scalar_prefetch=1,
            grid=(NB,),
            in_specs=[
                pl.BlockSpec((BM, D), lambda i, be: (i, 0)),
                pl.BlockSpec((1, D, D), lambda i, be: (be[i], 0, 0)),
                pl.BlockSpec((1, 1, D), lambda i, be: (be[i], 0, 0)),
            ],
            out_specs=pl.BlockSpec((BM, D), lambda i, be: (i, 0)),
        ),
        out_shape=jax.ShapeDtypeStruct((P, D), jnp.float32),
        compiler_params=pltpu.CompilerParams(
            dimension_semantics=("arbitrary",)),
    )(be, xs, aW, ab3)


def _combine_gather(ys, pos1, pos2):
    mesh = plsc.VectorSubcoreMesh(core_axis_name="c", subcore_axis_name="s")
    f = functools.partial(
        pl.kernel,
        mesh=mesh,
        out_type=[
            jax.ShapeDtypeStruct((T, D), jnp.float32),
            jax.ShapeDtypeStruct((T, D), jnp.float32),
        ],
        scratch_types=[
            pltpu.VMEM((NCHC, CHC), jnp.int32),
            pltpu.VMEM((NCHC, CHC), jnp.int32),
            pltpu.VMEM((CHC, D), jnp.float32),
            pltpu.VMEM((CHC, D), jnp.float32),
            pltpu.VMEM((CHC, D), jnp.float32),
            pltpu.VMEM((CHC, D), jnp.float32),
            pltpu.SemaphoreType.DMA,
            pltpu.SemaphoreType.DMA,
            pltpu.SemaphoreType.DMA,
        ],
    )(_k6_sc)
    return f(ys, pos1, pos2)


def _combine_log(r1, r2, g1, g2):
    return pl.pallas_call(
        _k7_body,
        grid=(NBLK,),
        in_specs=[
            pl.BlockSpec((BT, D), lambda i: (i, 0)),
            pl.BlockSpec((BT, D), lambda i: (i, 0)),
            pl.BlockSpec((BT, 1), lambda i: (i, 0)),
            pl.BlockSpec((BT, 1), lambda i: (i, 0)),
        ],
        out_specs=pl.BlockSpec((BT, D), lambda i: (i, 0)),
        out_shape=jax.ShapeDtypeStruct((T, D), jnp.float32),
        compiler_params=pltpu.CompilerParams(
            dimension_semantics=("arbitrary",)),
    )(r1, r2, g1, g2)


@jax.jit
def kernel(feature, x, gW1, gb1, gW2, gb2, aW, ab):
    (g1, g2, e1, e2, rank1, rank2, blk_excl, seg_start, be,
     _imp, loss2d) = _gating(feature, gW1, gb1, gW2, gb2)
    pos1, pos2 = _positions(e1, e2, rank1, rank2, blk_excl, seg_start)
    xs = _dispatch(x, pos1.reshape(NW, NCHD, CHD), pos2.reshape(NW, NCHD, CHD))
    ys = _expert_mm(xs, aW, ab.reshape(E, 1, D), be)
    r1, r2 = _combine_gather(ys, pos1.reshape(NW, NCHC, CHC),
                             pos2.reshape(NW, NCHC, CHC))
    y = _combine_log(r1, r2, g1, g2)
    return y, loss2d[0, 0]
